# baseline (speedup 1.0000x reference)
import dataclasses
import functools

import jax
import jax.numpy as jnp
from jax import lax
from jax.experimental import pallas as pl
from jax.experimental.pallas import tpu as pltpu
from jax.experimental.pallas import tpu_sc as plsc

N = 10000
NP = 10240
E = 160000
EP = 163840
D = 256
L = 3
K = 3
CH = 128
NC = 2
NS = 16
RT = NP // NS
EC = 32
F32 = jnp.float32

_mesh = plsc.VectorSubcoreMesh(core_axis_name="c", subcore_axis_name="s")

_sc_params = pltpu.CompilerParams()
if "needs_layout_passes" in pltpu.CompilerParams.__dataclass_fields__:
    _sc_params = dataclasses.replace(_sc_params, needs_layout_passes=False)


def _splat(vec_ref, i):
    idx = jnp.full((16,), 0, jnp.int32) + i
    return plsc.load_gather(vec_ref, [idx])


@functools.partial(
    pl.kernel,
    mesh=_mesh,
    compiler_params=_sc_params,
    out_type=(
        jax.ShapeDtypeStruct((NC, NP, 128), F32),
        jax.ShapeDtypeStruct((NC, NP, 128), F32),
    ),
    scratch_types=[
        pltpu.VMEM_SHARED((NP, 128), F32),
        pltpu.VMEM((2, CH), jnp.int32),
        pltpu.VMEM((2, CH), jnp.int32),
        pltpu.VMEM((2, CH), F32),
        pltpu.VMEM((2, CH, 128), F32),
        pltpu.VMEM((EC, 128), F32),
        pltpu.VMEM((EC,), F32),
        pltpu.VMEM((EC,), F32),
        pltpu.SemaphoreType.DMA,
        pltpu.SemaphoreType.DMA,
    ],
)
def _spmm_kernel(src_hbm, w_hbm, row_hbm, col_hbm, dinv_hbm, dinv2_hbm,
                 xk_hbm, uk_hbm,
                 acc_sh, row_v, col_v, w_v, gbuf, stage, d1_v, d2_v,
                 gsem0, gsem1):
    c = lax.axis_index("c")
    s = lax.axis_index("s")
    coff = c * NP
    gsems = (gsem0, gsem1)

    @pl.loop(0, EC)
    def _(r):
        for v in range(8):
            stage[r, pl.ds(v * 16, 16)] = jnp.zeros((16,), F32)

    @pl.loop(0, RT // EC)
    def _(z):
        pltpu.sync_copy(stage, acc_sh.at[pl.ds(s * RT + z * EC, EC)])

    plsc.subcore_barrier()

    ept = EP // NS
    nch = ept // CH
    base = s * ept

    def load_idx(g, d):
        off = base + g * CH
        pltpu.sync_copy(row_hbm.at[pl.ds(off, CH)], row_v.at[d])
        pltpu.sync_copy(col_hbm.at[pl.ds(off, CH)], col_v.at[d])
        pltpu.sync_copy(w_hbm.at[pl.ds(off, CH)], w_v.at[d])

        @pl.loop(0, CH // 16)
        def _(i):
            row_v[d, pl.ds(i * 16, 16)] = row_v[d, pl.ds(i * 16, 16)] + coff

    for d in range(2):
        load_idx(d, d)
        pltpu.async_copy(src_hbm.at[row_v.at[d]], gbuf.at[d], gsems[d])

    @pl.loop(0, nch, step=2)
    def _(g0):
        for d in range(2):
            g = g0 + d
            pltpu.make_async_copy(src_hbm.at[row_v.at[d]], gbuf.at[d],
                                  gsems[d]).wait()

            @plsc.parallel_loop(0, CH // 16, unroll=4)
            def _(gg):
                z16 = jnp.full((16,), 0, jnp.int32)
                for j in range(16):
                    e = gg * 16 + j
                    wv = plsc.load_gather(w_v, [z16 + d, z16 + e])
                    for v in range(8):
                        gbuf[d, e, pl.ds(v * 16, 16)] = (
                            gbuf[d, e, pl.ds(v * 16, 16)] * wv)

            pltpu.sync_copy(gbuf.at[d], acc_sh.at[col_v.at[d]], add=True)

            @pl.when(g + 2 < nch)
            def _():
                load_idx(g + 2, d)
                pltpu.async_copy(src_hbm.at[row_v.at[d]], gbuf.at[d], gsems[d])

    plsc.subcore_barrier()

    @pl.loop(0, RT // EC)
    def _(z):
        r0 = s * RT + z * EC
        pltpu.sync_copy(acc_sh.at[pl.ds(r0, EC)], stage)
        pltpu.sync_copy(dinv_hbm.at[pl.ds(r0, EC)], d1_v)
        pltpu.sync_copy(dinv2_hbm.at[pl.ds(r0, EC)], d2_v)

        @pl.loop(0, EC // 16)
        def _(gg):
            for j in range(16):
                r = gg * 16 + j
                dv1 = _splat(d1_v, r)
                dv2 = _splat(d2_v, r)
                for v in range(8):
                    gg16 = stage[r, pl.ds(v * 16, 16)]
                    gbuf[0, r, pl.ds(v * 16, 16)] = gg16 * dv1
                    stage[r, pl.ds(v * 16, 16)] = gg16 * dv2

        pltpu.sync_copy(gbuf.at[0, pl.ds(0, EC)], xk_hbm.at[c, pl.ds(r0, EC)])
        pltpu.sync_copy(stage, uk_hbm.at[c, pl.ds(r0, EC)])


def _prep_u0(x_split, dinv_n1):
    BN = 2048

    def body(x_ref, dv_ref, ou_ref):
        dcol = dv_ref[...]
        ou_ref[0] = x_ref[0] * dcol
        ou_ref[1] = x_ref[1] * dcol

    return pl.pallas_call(
        body,
        grid=(NP // BN,),
        in_specs=[
            pl.BlockSpec((NC, BN, 128), lambda j: (0, j, 0)),
            pl.BlockSpec((BN, 1), lambda j: (j, 0)),
        ],
        out_specs=pl.BlockSpec((NC, BN, 128), lambda j: (0, j, 0)),
        out_shape=jax.ShapeDtypeStruct((NC, NP, 128), F32),
    )(x_split, dinv_n1)


def _mm(xk, w2, p_prev):
    BN = 1024
    has_p = p_prev is not None

    def body(x0_ref, x1_ref, w_ref, *rest):
        if has_p:
            p_ref, o_ref = rest
        else:
            (o_ref,) = rest
        acc = jnp.dot(x0_ref[0], w_ref[0], preferred_element_type=F32,
                      precision=lax.Precision.HIGHEST)
        acc = acc + jnp.dot(x1_ref[0], w_ref[1], preferred_element_type=F32,
                            precision=lax.Precision.HIGHEST)
        if has_p:
            acc = acc + p_ref[...]
        o_ref[...] = acc

    in_specs = [
        pl.BlockSpec((1, BN, 128), lambda j: (0, j, 0)),
        pl.BlockSpec((1, BN, 128), lambda j: (1, j, 0)),
        pl.BlockSpec((NC, 128, D), lambda j: (0, 0, 0)),
    ]
    args = [xk, xk, w2]
    if has_p:
        in_specs.append(pl.BlockSpec((BN, D), lambda j: (j, 0)))
        args.append(p_prev)

    return pl.pallas_call(
        body,
        grid=(NP // BN,),
        in_specs=in_specs,
        out_specs=pl.BlockSpec((BN, D), lambda j: (j, 0)),
        out_shape=jax.ShapeDtypeStruct((NP, D), F32),
    )(*args)


def _ln_relu(x_split, p, bl, gl, betl, dinv_n1, want_u):
    BN = 512

    def body(x_ref, p_ref, b_ref, g_ref, bt_ref, dv_ref, o_ref, ou_ref=None):
        xf = jnp.concatenate([x_ref[0], x_ref[1]], axis=-1)
        y = xf + p_ref[...] + b_ref[...]
        m = jnp.mean(y, axis=-1, keepdims=True)
        yc = y - m
        var = jnp.mean(yc * yc, axis=-1, keepdims=True)
        y = yc * (1.0 / jnp.sqrt(var + 1e-5)) * g_ref[...] + bt_ref[...]
        xn = jnp.maximum(y, 0.0)
        o_ref[0] = xn[:, :128]
        o_ref[1] = xn[:, 128:]
        if ou_ref is not None:
            dcol = dv_ref[...]
            ou_ref[0] = xn[:, :128] * dcol
            ou_ref[1] = xn[:, 128:] * dcol

    out_specs = [pl.BlockSpec((NC, BN, 128), lambda j: (0, j, 0))]
    out_shape = [jax.ShapeDtypeStruct((NC, NP, 128), F32)]
    if want_u:
        out_specs.append(pl.BlockSpec((NC, BN, 128), lambda j: (0, j, 0)))
        out_shape.append(jax.ShapeDtypeStruct((NC, NP, 128), F32))

    res = pl.pallas_call(
        body,
        grid=(NP // BN,),
        in_specs=[
            pl.BlockSpec((NC, BN, 128), lambda j: (0, j, 0)),
            pl.BlockSpec((BN, D), lambda j: (j, 0)),
            pl.BlockSpec((1, D), lambda j: (0, 0)),
            pl.BlockSpec((1, D), lambda j: (0, 0)),
            pl.BlockSpec((1, D), lambda j: (0, 0)),
            pl.BlockSpec((BN, 1), lambda j: (j, 0)),
        ],
        out_specs=out_specs,
        out_shape=out_shape,
    )(x_split, p, bl, gl, betl, dinv_n1)
    return res if want_u else (res[0], None)


def kernel(node, edge_index, edge_attr, batch_ptr, W, b, gamma, beta):
    row = edge_index[0].astype(jnp.int32)
    col = edge_index[1].astype(jnp.int32)

    padn = EP - E
    ar = jnp.arange(padn, dtype=jnp.int32)
    row_p = jnp.concatenate([row, (ar * 61) % N])
    col_p = jnp.concatenate([col, N + (ar % (NP - N))])
    ea_p = jnp.concatenate([edge_attr, jnp.zeros((padn,), F32)])

    node_pad = jnp.zeros((NP, D), F32).at[:N].set(node)
    x = jnp.stack([node_pad[:, :128], node_pad[:, 128:]])

    onesN = jnp.ones((NP,), F32)
    ones_src = jnp.ones((NC * NP, 128), F32)
    degx, _ = _spmm_kernel(ones_src, ea_p, row_p, col_p, onesN, onesN)
    deg = degx[0, :, 0]
    dinv = jnp.where(deg > 0, 1.0 / jnp.sqrt(jnp.where(deg > 0, deg, 1.0)), 0.0)
    dinv2 = dinv * dinv
    dinv_n1 = dinv[:, None]

    u = _prep_u0(x, dinv_n1)

    for l in range(L):
        w2 = [jnp.stack([W[l, k][:, :128].T, W[l, k][:, 128:].T])
              for k in range(K + 1)]
        x1, u1 = _spmm_kernel(u.reshape(NC * NP, 128), ea_p, row_p, col_p,
                              dinv, dinv2)
        p = _mm(x, w2[0], None)
        x2, u2 = _spmm_kernel(u1.reshape(NC * NP, 128), ea_p, row_p, col_p,
                              dinv, dinv2)
        p = _mm(x1, w2[1], p)
        x3, _ = _spmm_kernel(u2.reshape(NC * NP, 128), ea_p, row_p, col_p,
                             dinv, dinv2)
        p = _mm(x2, w2[2], p)
        p = _mm(x3, w2[3], p)
        x, u = _ln_relu(x, p, b[l][None], gamma[l][None], beta[l][None],
                        dinv_n1, want_u=(l < L - 1))

    return jnp.transpose(x, (1, 0, 2)).reshape(NP, D)[:N]


# speedup vs baseline: 1.4043x; 1.4043x over previous
import dataclasses
import functools

import jax
import jax.numpy as jnp
from jax import lax
from jax.experimental import pallas as pl
from jax.experimental.pallas import tpu as pltpu
from jax.experimental.pallas import tpu_sc as plsc

N = 10000
NP = 10240
E = 160000
EP = 163840
D = 256
L = 3
K = 3
CH = 128
NC = 2
NS = 16
RT = NP // NS
EC = 32
F32 = jnp.float32

_mesh = plsc.VectorSubcoreMesh(core_axis_name="c", subcore_axis_name="s")

_sc_params = pltpu.CompilerParams()
if "needs_layout_passes" in pltpu.CompilerParams.__dataclass_fields__:
    _sc_params = dataclasses.replace(_sc_params, needs_layout_passes=False)


def _splat(vec_ref, i):
    idx = jnp.full((16,), 0, jnp.int32) + i
    return plsc.load_gather(vec_ref, [idx])


@functools.partial(
    pl.kernel,
    mesh=_mesh,
    compiler_params=_sc_params,
    out_type=(
        jax.ShapeDtypeStruct((NC, NP, 128), F32),
        jax.ShapeDtypeStruct((NC, NP, 128), F32),
    ),
    scratch_types=[
        pltpu.VMEM_SHARED((NP, 128), F32),
        pltpu.VMEM((2, 4, CH), jnp.int32),
        pltpu.VMEM((2, 4, CH), jnp.int32),
        pltpu.VMEM((2, 4, CH), F32),
        pltpu.VMEM((2, CH, 128), F32),
        pltpu.VMEM((EC, 128), F32),
        pltpu.VMEM((EC,), F32),
        pltpu.VMEM((EC,), F32),
        pltpu.SemaphoreType.DMA,
        pltpu.SemaphoreType.DMA,
    ],
)
def _spmm_kernel(src_hbm, w_hbm, row_hbm, col_hbm, dinv_hbm, dinv2_hbm,
                 xk_hbm, uk_hbm,
                 acc_sh, row_v, col_v, w_v, gbuf, stage, d1_v, d2_v,
                 gsem0, gsem1):
    c = lax.axis_index("c")
    s = lax.axis_index("s")
    coff = c * NP
    gsems = (gsem0, gsem1)

    @pl.loop(0, EC)
    def _(r):
        for v in range(8):
            stage[r, pl.ds(v * 16, 16)] = jnp.zeros((16,), F32)

    @pl.loop(0, RT // EC)
    def _(z):
        pltpu.sync_copy(stage, acc_sh.at[pl.ds(s * RT + z * EC, EC)])

    plsc.subcore_barrier()

    ept = EP // NS
    nch = ept // CH
    nsc = nch // 4
    base = s * ept

    cbase = s * (ept // CH)

    def load_super(sc_i, p):
        coff4 = cbase + sc_i * 4
        pltpu.sync_copy(row_hbm.at[pl.ds(coff4, 4)], row_v.at[p])
        pltpu.sync_copy(col_hbm.at[pl.ds(coff4, 4)], col_v.at[p])
        pltpu.sync_copy(w_hbm.at[pl.ds(coff4, 4)], w_v.at[p])

        @pl.loop(0, 4)
        def _(q2):
            @pl.loop(0, CH // 16)
            def _(i):
                row_v[p, q2, pl.ds(i * 16, 16)] = (
                    row_v[p, q2, pl.ds(i * 16, 16)] + coff)

    load_super(0, 0)
    for d in range(2):
        pltpu.async_copy(src_hbm.at[row_v.at[0, d]], gbuf.at[d], gsems[d])

    @pl.loop(0, nsc)
    def _(sc_i):
        p = lax.rem(sc_i, 2)
        pnext = 1 - p

        @pl.when(sc_i + 1 < nsc)
        def _():
            load_super(sc_i + 1, pnext)

        for q in range(4):
            g = sc_i * 4 + q
            d = q % 2
            pltpu.make_async_copy(src_hbm.at[row_v.at[p, q]], gbuf.at[d],
                                  gsems[d]).wait()

            @plsc.parallel_loop(0, CH // 16, unroll=2)
            def _(gg):
                z16 = jnp.full((16,), 0, jnp.int32)
                for j in range(16):
                    e = gg * 16 + j
                    wv = plsc.load_gather(w_v, [z16 + p, z16 + q, z16 + e])
                    for v in range(8):
                        gbuf[d, e, pl.ds(v * 16, 16)] = (
                            gbuf[d, e, pl.ds(v * 16, 16)] * wv)

            pltpu.sync_copy(gbuf.at[d], acc_sh.at[col_v.at[p, q]], add=True)

            @pl.when(g + 2 < nch)
            def _():
                if q < 2:
                    pltpu.async_copy(src_hbm.at[row_v.at[p, q + 2]],
                                     gbuf.at[d], gsems[d])
                else:
                    pltpu.async_copy(src_hbm.at[row_v.at[pnext, q - 2]],
                                     gbuf.at[d], gsems[d])

    plsc.subcore_barrier()

    @pl.loop(0, RT // EC)
    def _(z):
        r0 = s * RT + z * EC
        pltpu.sync_copy(acc_sh.at[pl.ds(r0, EC)], stage)
        pltpu.sync_copy(dinv_hbm.at[pl.ds(r0, EC)], d1_v)
        pltpu.sync_copy(dinv2_hbm.at[pl.ds(r0, EC)], d2_v)

        @pl.loop(0, EC // 16)
        def _(gg):
            for j in range(16):
                r = gg * 16 + j
                dv1 = _splat(d1_v, r)
                dv2 = _splat(d2_v, r)
                for v in range(8):
                    gg16 = stage[r, pl.ds(v * 16, 16)]
                    gbuf[0, r, pl.ds(v * 16, 16)] = gg16 * dv1
                    stage[r, pl.ds(v * 16, 16)] = gg16 * dv2

        pltpu.sync_copy(gbuf.at[0, pl.ds(0, EC)], xk_hbm.at[c, pl.ds(r0, EC)])
        pltpu.sync_copy(stage, uk_hbm.at[c, pl.ds(r0, EC)])


def _prep_u0(x_split, dinv_n1):
    BN = 2048

    def body(x_ref, dv_ref, ou_ref):
        dcol = dv_ref[...]
        ou_ref[0] = x_ref[0] * dcol
        ou_ref[1] = x_ref[1] * dcol

    return pl.pallas_call(
        body,
        grid=(NP // BN,),
        in_specs=[
            pl.BlockSpec((NC, BN, 128), lambda j: (0, j, 0)),
            pl.BlockSpec((BN, 1), lambda j: (j, 0)),
        ],
        out_specs=pl.BlockSpec((NC, BN, 128), lambda j: (0, j, 0)),
        out_shape=jax.ShapeDtypeStruct((NC, NP, 128), F32),
    )(x_split, dinv_n1)


def _mm(xk, w2, p_prev):
    BN = 1024
    has_p = p_prev is not None

    def body(x0_ref, x1_ref, w_ref, *rest):
        if has_p:
            p_ref, o_ref = rest
        else:
            (o_ref,) = rest
        acc = jnp.dot(x0_ref[0], w_ref[0], preferred_element_type=F32,
                      precision=lax.Precision.HIGHEST)
        acc = acc + jnp.dot(x1_ref[0], w_ref[1], preferred_element_type=F32,
                            precision=lax.Precision.HIGHEST)
        if has_p:
            acc = acc + p_ref[...]
        o_ref[...] = acc

    in_specs = [
        pl.BlockSpec((1, BN, 128), lambda j: (0, j, 0)),
        pl.BlockSpec((1, BN, 128), lambda j: (1, j, 0)),
        pl.BlockSpec((NC, 128, D), lambda j: (0, 0, 0)),
    ]
    args = [xk, xk, w2]
    if has_p:
        in_specs.append(pl.BlockSpec((BN, D), lambda j: (j, 0)))
        args.append(p_prev)

    return pl.pallas_call(
        body,
        grid=(NP // BN,),
        in_specs=in_specs,
        out_specs=pl.BlockSpec((BN, D), lambda j: (j, 0)),
        out_shape=jax.ShapeDtypeStruct((NP, D), F32),
    )(*args)


def _ln_relu(x_split, p, bl, gl, betl, dinv_n1, want_u):
    BN = 512

    def body(x_ref, p_ref, b_ref, g_ref, bt_ref, dv_ref, o_ref, ou_ref=None):
        xf = jnp.concatenate([x_ref[0], x_ref[1]], axis=-1)
        y = xf + p_ref[...] + b_ref[...]
        m = jnp.mean(y, axis=-1, keepdims=True)
        yc = y - m
        var = jnp.mean(yc * yc, axis=-1, keepdims=True)
        y = yc * (1.0 / jnp.sqrt(var + 1e-5)) * g_ref[...] + bt_ref[...]
        xn = jnp.maximum(y, 0.0)
        o_ref[0] = xn[:, :128]
        o_ref[1] = xn[:, 128:]
        if ou_ref is not None:
            dcol = dv_ref[...]
            ou_ref[0] = xn[:, :128] * dcol
            ou_ref[1] = xn[:, 128:] * dcol

    out_specs = [pl.BlockSpec((NC, BN, 128), lambda j: (0, j, 0))]
    out_shape = [jax.ShapeDtypeStruct((NC, NP, 128), F32)]
    if want_u:
        out_specs.append(pl.BlockSpec((NC, BN, 128), lambda j: (0, j, 0)))
        out_shape.append(jax.ShapeDtypeStruct((NC, NP, 128), F32))

    res = pl.pallas_call(
        body,
        grid=(NP // BN,),
        in_specs=[
            pl.BlockSpec((NC, BN, 128), lambda j: (0, j, 0)),
            pl.BlockSpec((BN, D), lambda j: (j, 0)),
            pl.BlockSpec((1, D), lambda j: (0, 0)),
            pl.BlockSpec((1, D), lambda j: (0, 0)),
            pl.BlockSpec((1, D), lambda j: (0, 0)),
            pl.BlockSpec((BN, 1), lambda j: (j, 0)),
        ],
        out_specs=out_specs,
        out_shape=out_shape,
    )(x_split, p, bl, gl, betl, dinv_n1)
    return res if want_u else (res[0], None)


def kernel(node, edge_index, edge_attr, batch_ptr, W, b, gamma, beta):
    row = edge_index[0].astype(jnp.int32)
    col = edge_index[1].astype(jnp.int32)

    padn = EP - E
    ar = jnp.arange(padn, dtype=jnp.int32)
    row_p = jnp.concatenate([row, (ar * 61) % N]).reshape(EP // CH, CH)
    col_p = jnp.concatenate([col, N + (ar % (NP - N))]).reshape(EP // CH, CH)
    ea_p = jnp.concatenate([edge_attr,
                            jnp.zeros((padn,), F32)]).reshape(EP // CH, CH)

    node_pad = jnp.zeros((NP, D), F32).at[:N].set(node)
    x = jnp.stack([node_pad[:, :128], node_pad[:, 128:]])

    onesN = jnp.ones((NP,), F32)
    ones_src = jnp.ones((NC * NP, 128), F32)
    degx, _ = _spmm_kernel(ones_src, ea_p, row_p, col_p, onesN, onesN)
    deg = degx[0, :, 0]
    dinv = jnp.where(deg > 0, 1.0 / jnp.sqrt(jnp.where(deg > 0, deg, 1.0)), 0.0)
    dinv2 = dinv * dinv
    dinv_n1 = dinv[:, None]

    u = _prep_u0(x, dinv_n1)

    for l in range(L):
        w2 = [jnp.stack([W[l, k][:, :128].T, W[l, k][:, 128:].T])
              for k in range(K + 1)]
        x1, u1 = _spmm_kernel(u.reshape(NC * NP, 128), ea_p, row_p, col_p,
                              dinv, dinv2)
        p = _mm(x, w2[0], None)
        x2, u2 = _spmm_kernel(u1.reshape(NC * NP, 128), ea_p, row_p, col_p,
                              dinv, dinv2)
        p = _mm(x1, w2[1], p)
        x3, _ = _spmm_kernel(u2.reshape(NC * NP, 128), ea_p, row_p, col_p,
                             dinv, dinv2)
        p = _mm(x2, w2[2], p)
        p = _mm(x3, w2[3], p)
        x, u = _ln_relu(x, p, b[l][None], gamma[l][None], beta[l][None],
                        dinv_n1, want_u=(l < L - 1))

    return jnp.transpose(x, (1, 0, 2)).reshape(NP, D)[:N]


# speedup vs baseline: 1.4715x; 1.0479x over previous
import dataclasses
import functools

import jax
import jax.numpy as jnp
from jax import lax
from jax.experimental import pallas as pl
from jax.experimental.pallas import tpu as pltpu
from jax.experimental.pallas import tpu_sc as plsc

N = 10000
NP = 10240
E = 160000
EP = 163840
D = 256
L = 3
K = 3
CH = 128
NC = 2
NS = 16
RT = NP // NS
EC = 32
F32 = jnp.float32

_mesh = plsc.VectorSubcoreMesh(core_axis_name="c", subcore_axis_name="s")

_sc_params = pltpu.CompilerParams()
if "needs_layout_passes" in pltpu.CompilerParams.__dataclass_fields__:
    _sc_params = dataclasses.replace(_sc_params, needs_layout_passes=False)


def _splat(vec_ref, i):
    idx = jnp.full((16,), 0, jnp.int32) + i
    return plsc.load_gather(vec_ref, [idx])


@functools.partial(
    pl.kernel,
    mesh=_mesh,
    compiler_params=_sc_params,
    out_type=(
        jax.ShapeDtypeStruct((NC, NP, 128), F32),
        jax.ShapeDtypeStruct((NC, NP, 128), F32),
    ),
    scratch_types=[
        pltpu.VMEM_SHARED((NP, 128), F32),
        pltpu.VMEM((2, 4, CH), jnp.int32),
        pltpu.VMEM((2, 4, CH), jnp.int32),
        pltpu.VMEM((2, 4, CH), F32),
        pltpu.VMEM((2, CH, 128), F32),
        pltpu.VMEM((EC, 128), F32),
        pltpu.VMEM((EC,), F32),
        pltpu.VMEM((EC,), F32),
        pltpu.SemaphoreType.DMA,
        pltpu.SemaphoreType.DMA,
    ],
)
def _spmm_kernel(src_hbm, w_hbm, row_hbm, col_hbm, dinv_hbm, dinv2_hbm,
                 xk_hbm, uk_hbm,
                 acc_sh, row_v, col_v, w_v, gbuf, stage, d1_v, d2_v,
                 gsem0, gsem1):
    c = lax.axis_index("c")
    s = lax.axis_index("s")
    coff = c * NP
    gsems = (gsem0, gsem1)

    @pl.loop(0, EC)
    def _(r):
        for v in range(8):
            stage[r, pl.ds(v * 16, 16)] = jnp.zeros((16,), F32)

    @pl.loop(0, RT // EC)
    def _(z):
        pltpu.sync_copy(stage, acc_sh.at[pl.ds(s * RT + z * EC, EC)])

    plsc.subcore_barrier()

    ept = EP // NS
    nch = ept // CH
    nsc = nch // 4
    base = s * ept

    cbase = s * (ept // CH)

    def load_super(sc_i, p):
        coff4 = cbase + sc_i * 4
        pltpu.sync_copy(row_hbm.at[pl.ds(coff4, 4)], row_v.at[p])
        pltpu.sync_copy(col_hbm.at[pl.ds(coff4, 4)], col_v.at[p])
        pltpu.sync_copy(w_hbm.at[pl.ds(coff4, 4)], w_v.at[p])

        @pl.loop(0, 4)
        def _(q2):
            @pl.loop(0, CH // 16)
            def _(i):
                row_v[p, q2, pl.ds(i * 16, 16)] = (
                    row_v[p, q2, pl.ds(i * 16, 16)] + coff)

    load_super(0, 0)
    for d in range(2):
        pltpu.async_copy(src_hbm.at[row_v.at[0, d]], gbuf.at[d], gsems[d])

    @pl.loop(0, nsc)
    def _(sc_i):
        p = lax.rem(sc_i, 2)
        pnext = 1 - p

        @pl.when(sc_i + 1 < nsc)
        def _():
            load_super(sc_i + 1, pnext)

        for q in range(4):
            g = sc_i * 4 + q
            d = q % 2
            pltpu.make_async_copy(src_hbm.at[row_v.at[p, q]], gbuf.at[d],
                                  gsems[d]).wait()

            @plsc.parallel_loop(0, CH // 16, unroll=2)
            def _(gg):
                z16 = jnp.full((16,), 0, jnp.int32)
                for j in range(16):
                    e = gg * 16 + j
                    wv = plsc.load_gather(w_v, [z16 + p, z16 + q, z16 + e])
                    for v in range(8):
                        gbuf[d, e, pl.ds(v * 16, 16)] = (
                            gbuf[d, e, pl.ds(v * 16, 16)] * wv)

            pltpu.sync_copy(gbuf.at[d], acc_sh.at[col_v.at[p, q]], add=True)

            @pl.when(g + 2 < nch)
            def _():
                if q < 2:
                    pltpu.async_copy(src_hbm.at[row_v.at[p, q + 2]],
                                     gbuf.at[d], gsems[d])
                else:
                    pltpu.async_copy(src_hbm.at[row_v.at[pnext, q - 2]],
                                     gbuf.at[d], gsems[d])

    plsc.subcore_barrier()

    @pl.loop(0, RT // EC)
    def _(z):
        r0 = s * RT + z * EC
        pltpu.sync_copy(acc_sh.at[pl.ds(r0, EC)], stage)
        pltpu.sync_copy(dinv_hbm.at[pl.ds(r0, EC)], d1_v)
        pltpu.sync_copy(dinv2_hbm.at[pl.ds(r0, EC)], d2_v)

        @pl.loop(0, EC // 16)
        def _(gg):
            for j in range(16):
                r = gg * 16 + j
                dv1 = _splat(d1_v, r)
                dv2 = _splat(d2_v, r)
                for v in range(8):
                    gg16 = stage[r, pl.ds(v * 16, 16)]
                    gbuf[0, r, pl.ds(v * 16, 16)] = gg16 * dv1
                    stage[r, pl.ds(v * 16, 16)] = gg16 * dv2

        pltpu.sync_copy(gbuf.at[0, pl.ds(0, EC)], xk_hbm.at[c, pl.ds(r0, EC)])
        pltpu.sync_copy(stage, uk_hbm.at[c, pl.ds(r0, EC)])


@functools.partial(
    pl.kernel,
    mesh=_mesh,
    compiler_params=_sc_params,
    out_type=jax.ShapeDtypeStruct((NC, NP, 128), F32),
    scratch_types=[
        pltpu.VMEM_SHARED((NP, 128), F32),
        pltpu.VMEM((1, CH), jnp.int32),
        pltpu.VMEM((1, CH), F32),
        pltpu.VMEM((CH, 128), F32),
        pltpu.VMEM((EC, 128), F32),
    ],
)
def _deg_kernel(col_hbm, ea_hbm, deg_hbm, acc_sh, col_v, w_v, sbuf, stage):
    c = lax.axis_index("c")
    s = lax.axis_index("s")

    @pl.loop(0, EC)
    def _(r):
        for v in range(8):
            stage[r, pl.ds(v * 16, 16)] = jnp.zeros((16,), F32)

    @pl.loop(0, RT // EC)
    def _(z):
        pltpu.sync_copy(stage, acc_sh.at[pl.ds(s * RT + z * EC, EC)])

    plsc.subcore_barrier()

    nch_all = EP // CH
    cbase = (c * NS + s) * (nch_all // (NC * NS))

    @pl.loop(0, nch_all // (NC * NS))
    def _(g):
        pltpu.sync_copy(col_hbm.at[pl.ds(cbase + g, 1)], col_v)
        pltpu.sync_copy(ea_hbm.at[pl.ds(cbase + g, 1)], w_v)

        @plsc.parallel_loop(0, CH // 16, unroll=2)
        def _(gg):
            z16 = jnp.full((16,), 0, jnp.int32)
            for j in range(16):
                e = gg * 16 + j
                wv = plsc.load_gather(w_v, [z16, z16 + e])
                for v in range(8):
                    sbuf[e, pl.ds(v * 16, 16)] = wv

        pltpu.sync_copy(sbuf, acc_sh.at[col_v.at[0]], add=True)

    plsc.subcore_barrier()
    pltpu.sync_copy(acc_sh.at[pl.ds(s * RT, RT)], deg_hbm.at[c, pl.ds(s * RT, RT)])


def _prep_u0(x_split, dinv_n1):
    BN = 2048

    def body(x_ref, dv_ref, ou_ref):
        dcol = dv_ref[...]
        ou_ref[0] = x_ref[0] * dcol
        ou_ref[1] = x_ref[1] * dcol

    return pl.pallas_call(
        body,
        grid=(NP // BN,),
        in_specs=[
            pl.BlockSpec((NC, BN, 128), lambda j: (0, j, 0)),
            pl.BlockSpec((BN, 1), lambda j: (j, 0)),
        ],
        out_specs=pl.BlockSpec((NC, BN, 128), lambda j: (0, j, 0)),
        out_shape=jax.ShapeDtypeStruct((NC, NP, 128), F32),
    )(x_split, dinv_n1)


def _mm(xk, w2, p_prev):
    BN = 1024
    has_p = p_prev is not None

    def body(x0_ref, x1_ref, w_ref, *rest):
        if has_p:
            p_ref, o_ref = rest
        else:
            (o_ref,) = rest
        acc = jnp.dot(x0_ref[0], w_ref[0], preferred_element_type=F32,
                      precision=lax.Precision.HIGHEST)
        acc = acc + jnp.dot(x1_ref[0], w_ref[1], preferred_element_type=F32,
                            precision=lax.Precision.HIGHEST)
        if has_p:
            acc = acc + p_ref[...]
        o_ref[...] = acc

    in_specs = [
        pl.BlockSpec((1, BN, 128), lambda j: (0, j, 0)),
        pl.BlockSpec((1, BN, 128), lambda j: (1, j, 0)),
        pl.BlockSpec((NC, 128, D), lambda j: (0, 0, 0)),
    ]
    args = [xk, xk, w2]
    if has_p:
        in_specs.append(pl.BlockSpec((BN, D), lambda j: (j, 0)))
        args.append(p_prev)

    return pl.pallas_call(
        body,
        grid=(NP // BN,),
        in_specs=in_specs,
        out_specs=pl.BlockSpec((BN, D), lambda j: (j, 0)),
        out_shape=jax.ShapeDtypeStruct((NP, D), F32),
    )(*args)


def _ln_relu(x_split, p, bl, gl, betl, dinv_n1, want_u):
    BN = 512

    def body(x_ref, p_ref, b_ref, g_ref, bt_ref, dv_ref, o_ref, ou_ref=None):
        xf = jnp.concatenate([x_ref[0], x_ref[1]], axis=-1)
        y = xf + p_ref[...] + b_ref[...]
        m = jnp.mean(y, axis=-1, keepdims=True)
        yc = y - m
        var = jnp.mean(yc * yc, axis=-1, keepdims=True)
        y = yc * (1.0 / jnp.sqrt(var + 1e-5)) * g_ref[...] + bt_ref[...]
        xn = jnp.maximum(y, 0.0)
        o_ref[0] = xn[:, :128]
        o_ref[1] = xn[:, 128:]
        if ou_ref is not None:
            dcol = dv_ref[...]
            ou_ref[0] = xn[:, :128] * dcol
            ou_ref[1] = xn[:, 128:] * dcol

    out_specs = [pl.BlockSpec((NC, BN, 128), lambda j: (0, j, 0))]
    out_shape = [jax.ShapeDtypeStruct((NC, NP, 128), F32)]
    if want_u:
        out_specs.append(pl.BlockSpec((NC, BN, 128), lambda j: (0, j, 0)))
        out_shape.append(jax.ShapeDtypeStruct((NC, NP, 128), F32))

    res = pl.pallas_call(
        body,
        grid=(NP // BN,),
        in_specs=[
            pl.BlockSpec((NC, BN, 128), lambda j: (0, j, 0)),
            pl.BlockSpec((BN, D), lambda j: (j, 0)),
            pl.BlockSpec((1, D), lambda j: (0, 0)),
            pl.BlockSpec((1, D), lambda j: (0, 0)),
            pl.BlockSpec((1, D), lambda j: (0, 0)),
            pl.BlockSpec((BN, 1), lambda j: (j, 0)),
        ],
        out_specs=out_specs,
        out_shape=out_shape,
    )(x_split, p, bl, gl, betl, dinv_n1)
    return res if want_u else (res[0], None)


def kernel(node, edge_index, edge_attr, batch_ptr, W, b, gamma, beta):
    row = edge_index[0].astype(jnp.int32)
    col = edge_index[1].astype(jnp.int32)

    padn = EP - E
    ar = jnp.arange(padn, dtype=jnp.int32)
    row_p = jnp.concatenate([row, (ar * 61) % N]).reshape(EP // CH, CH)
    col_p = jnp.concatenate([col, N + (ar % (NP - N))]).reshape(EP // CH, CH)
    ea_p = jnp.concatenate([edge_attr,
                            jnp.zeros((padn,), F32)]).reshape(EP // CH, CH)

    node_pad = jnp.zeros((NP, D), F32).at[:N].set(node)
    x = jnp.stack([node_pad[:, :128], node_pad[:, 128:]])

    degx = _deg_kernel(col_p, ea_p)
    deg = degx[0, :, 0] + degx[1, :, 0]
    dinv = jnp.where(deg > 0, 1.0 / jnp.sqrt(jnp.where(deg > 0, deg, 1.0)), 0.0)
    dinv2 = dinv * dinv
    dinv_n1 = dinv[:, None]

    u = _prep_u0(x, dinv_n1)

    for l in range(L):
        w2 = [jnp.stack([W[l, k][:, :128].T, W[l, k][:, 128:].T])
              for k in range(K + 1)]
        x1, u1 = _spmm_kernel(u.reshape(NC * NP, 128), ea_p, row_p, col_p,
                              dinv, dinv2)
        p = _mm(x, w2[0], None)
        x2, u2 = _spmm_kernel(u1.reshape(NC * NP, 128), ea_p, row_p, col_p,
                              dinv, dinv2)
        p = _mm(x1, w2[1], p)
        x3, _ = _spmm_kernel(u2.reshape(NC * NP, 128), ea_p, row_p, col_p,
                             dinv, dinv2)
        p = _mm(x2, w2[2], p)
        p = _mm(x3, w2[3], p)
        x, u = _ln_relu(x, p, b[l][None], gamma[l][None], beta[l][None],
                        dinv_n1, want_u=(l < L - 1))

    return jnp.transpose(x, (1, 0, 2)).reshape(NP, D)[:N]


# speedup vs baseline: 1.5853x; 1.0773x over previous
import dataclasses
import functools

import jax
import jax.numpy as jnp
from jax import lax
from jax.experimental import pallas as pl
from jax.experimental.pallas import tpu as pltpu
from jax.experimental.pallas import tpu_sc as plsc

N = 10000
NP = 10240
E = 160000
EP = 163840
D = 256
L = 3
K = 3
CH = 128
NC = 2
NS = 16
RT = NP // NS
EC = 64
F32 = jnp.float32

_mesh = plsc.VectorSubcoreMesh(core_axis_name="c", subcore_axis_name="s")

_sc_params = pltpu.CompilerParams()
if "needs_layout_passes" in pltpu.CompilerParams.__dataclass_fields__:
    _sc_params = dataclasses.replace(_sc_params, needs_layout_passes=False)


def _splat(vec_ref, i):
    idx = jnp.full((16,), 0, jnp.int32) + i
    return plsc.load_gather(vec_ref, [idx])


@functools.partial(
    pl.kernel,
    mesh=_mesh,
    compiler_params=_sc_params,
    out_type=(
        jax.ShapeDtypeStruct((NC, NP, 128), F32),
        jax.ShapeDtypeStruct((NC, NP, 128), F32),
    ),
    scratch_types=[
        pltpu.VMEM_SHARED((NP, 128), F32),
        pltpu.VMEM((2, 4, CH), jnp.int32),
        pltpu.VMEM((2, 4, CH), jnp.int32),
        pltpu.VMEM((2, 4, CH), F32),
        pltpu.VMEM((2, CH, 128), F32),
        pltpu.VMEM((EC, 128), F32),
        pltpu.VMEM((RT,), F32),
        pltpu.VMEM((RT,), F32),
        pltpu.SemaphoreType.DMA,
        pltpu.SemaphoreType.DMA,
    ],
)
def _spmm_kernel(src_hbm, w_hbm, row_hbm, col_hbm, dinv_hbm, dinv2_hbm,
                 xk_hbm, uk_hbm,
                 acc_sh, row_v, col_v, w_v, gbuf, stage, d1_v, d2_v,
                 gsem0, gsem1):
    c = lax.axis_index("c")
    s = lax.axis_index("s")
    coff = c * NP
    gsems = (gsem0, gsem1)

    @pl.loop(0, EC)
    def _(r):
        for v in range(8):
            stage[r, pl.ds(v * 16, 16)] = jnp.zeros((16,), F32)

    @pl.loop(0, RT // EC)
    def _(z):
        pltpu.sync_copy(stage, acc_sh.at[pl.ds(s * RT + z * EC, EC)])

    plsc.subcore_barrier()

    ept = EP // NS
    nch = ept // CH
    nsc = nch // 4
    base = s * ept

    cbase = s * (ept // CH)

    def load_super(sc_i, p):
        coff4 = cbase + sc_i * 4
        pltpu.sync_copy(row_hbm.at[pl.ds(coff4, 4)], row_v.at[p])
        pltpu.sync_copy(col_hbm.at[pl.ds(coff4, 4)], col_v.at[p])
        pltpu.sync_copy(w_hbm.at[pl.ds(coff4, 4)], w_v.at[p])

        @pl.loop(0, 4)
        def _(q2):
            @pl.loop(0, CH // 16)
            def _(i):
                row_v[p, q2, pl.ds(i * 16, 16)] = (
                    row_v[p, q2, pl.ds(i * 16, 16)] + coff)

    load_super(0, 0)
    for d in range(2):
        pltpu.async_copy(src_hbm.at[row_v.at[0, d]], gbuf.at[d], gsems[d])

    @pl.loop(0, nsc)
    def _(sc_i):
        p = lax.rem(sc_i, 2)
        pnext = 1 - p

        @pl.when(sc_i + 1 < nsc)
        def _():
            load_super(sc_i + 1, pnext)

        for q in range(4):
            g = sc_i * 4 + q
            d = q % 2
            pltpu.make_async_copy(src_hbm.at[row_v.at[p, q]], gbuf.at[d],
                                  gsems[d]).wait()

            @plsc.parallel_loop(0, CH // 16, unroll=2)
            def _(gg):
                z16 = jnp.full((16,), 0, jnp.int32)
                for j in range(16):
                    e = gg * 16 + j
                    wv = plsc.load_gather(w_v, [z16 + p, z16 + q, z16 + e])
                    for v in range(8):
                        gbuf[d, e, pl.ds(v * 16, 16)] = (
                            gbuf[d, e, pl.ds(v * 16, 16)] * wv)

            pltpu.sync_copy(gbuf.at[d], acc_sh.at[col_v.at[p, q]], add=True)

            @pl.when(g + 2 < nch)
            def _():
                if q < 2:
                    pltpu.async_copy(src_hbm.at[row_v.at[p, q + 2]],
                                     gbuf.at[d], gsems[d])
                else:
                    pltpu.async_copy(src_hbm.at[row_v.at[pnext, q - 2]],
                                     gbuf.at[d], gsems[d])

    plsc.subcore_barrier()

    pltpu.sync_copy(dinv_hbm.at[pl.ds(s * RT, RT)], d1_v)
    pltpu.sync_copy(dinv2_hbm.at[pl.ds(s * RT, RT)], d2_v)

    @pl.loop(0, RT // EC)
    def _(z):
        r0 = s * RT + z * EC
        pltpu.sync_copy(acc_sh.at[pl.ds(r0, EC)], stage)

        @plsc.parallel_loop(0, EC // 16, unroll=2)
        def _(gg):
            for j in range(16):
                r = gg * 16 + j
                dv1 = _splat(d1_v, z * EC + r)
                dv2 = _splat(d2_v, z * EC + r)
                for v in range(8):
                    gg16 = stage[r, pl.ds(v * 16, 16)]
                    gbuf[0, r, pl.ds(v * 16, 16)] = gg16 * dv1
                    stage[r, pl.ds(v * 16, 16)] = gg16 * dv2

        pltpu.sync_copy(gbuf.at[0, pl.ds(0, EC)], xk_hbm.at[c, pl.ds(r0, EC)])
        pltpu.sync_copy(stage, uk_hbm.at[c, pl.ds(r0, EC)])


@functools.partial(
    pl.kernel,
    mesh=_mesh,
    compiler_params=_sc_params,
    out_type=jax.ShapeDtypeStruct((NC, NP, 128), F32),
    scratch_types=[
        pltpu.VMEM_SHARED((NP, 128), F32),
        pltpu.VMEM((1, CH), jnp.int32),
        pltpu.VMEM((1, CH), F32),
        pltpu.VMEM((CH, 128), F32),
        pltpu.VMEM((EC, 128), F32),
    ],
)
def _deg_kernel(col_hbm, ea_hbm, deg_hbm, acc_sh, col_v, w_v, sbuf, stage):
    c = lax.axis_index("c")
    s = lax.axis_index("s")

    @pl.loop(0, EC)
    def _(r):
        for v in range(8):
            stage[r, pl.ds(v * 16, 16)] = jnp.zeros((16,), F32)

    @pl.loop(0, RT // EC)
    def _(z):
        pltpu.sync_copy(stage, acc_sh.at[pl.ds(s * RT + z * EC, EC)])

    plsc.subcore_barrier()

    nch_all = EP // CH
    cbase = (c * NS + s) * (nch_all // (NC * NS))

    @pl.loop(0, nch_all // (NC * NS))
    def _(g):
        pltpu.sync_copy(col_hbm.at[pl.ds(cbase + g, 1)], col_v)
        pltpu.sync_copy(ea_hbm.at[pl.ds(cbase + g, 1)], w_v)

        @plsc.parallel_loop(0, CH // 16, unroll=2)
        def _(gg):
            z16 = jnp.full((16,), 0, jnp.int32)
            for j in range(16):
                e = gg * 16 + j
                wv = plsc.load_gather(w_v, [z16, z16 + e])
                for v in range(8):
                    sbuf[e, pl.ds(v * 16, 16)] = wv

        pltpu.sync_copy(sbuf, acc_sh.at[col_v.at[0]], add=True)

    plsc.subcore_barrier()
    pltpu.sync_copy(acc_sh.at[pl.ds(s * RT, RT)], deg_hbm.at[c, pl.ds(s * RT, RT)])


def _prep_u0(x_split, dinv_n1):
    BN = 2048

    def body(x_ref, dv_ref, ou_ref):
        dcol = dv_ref[...]
        ou_ref[0] = x_ref[0] * dcol
        ou_ref[1] = x_ref[1] * dcol

    return pl.pallas_call(
        body,
        grid=(NP // BN,),
        in_specs=[
            pl.BlockSpec((NC, BN, 128), lambda j: (0, j, 0)),
            pl.BlockSpec((BN, 1), lambda j: (j, 0)),
        ],
        out_specs=pl.BlockSpec((NC, BN, 128), lambda j: (0, j, 0)),
        out_shape=jax.ShapeDtypeStruct((NC, NP, 128), F32),
    )(x_split, dinv_n1)


def _mm(xk, w2, p_prev):
    BN = 1024
    has_p = p_prev is not None

    def body(x0_ref, x1_ref, w_ref, *rest):
        if has_p:
            p_ref, o_ref = rest
        else:
            (o_ref,) = rest
        acc = jnp.dot(x0_ref[0], w_ref[0], preferred_element_type=F32,
                      precision=lax.Precision.HIGHEST)
        acc = acc + jnp.dot(x1_ref[0], w_ref[1], preferred_element_type=F32,
                            precision=lax.Precision.HIGHEST)
        if has_p:
            acc = acc + p_ref[...]
        o_ref[...] = acc

    in_specs = [
        pl.BlockSpec((1, BN, 128), lambda j: (0, j, 0)),
        pl.BlockSpec((1, BN, 128), lambda j: (1, j, 0)),
        pl.BlockSpec((NC, 128, D), lambda j: (0, 0, 0)),
    ]
    args = [xk, xk, w2]
    if has_p:
        in_specs.append(pl.BlockSpec((BN, D), lambda j: (j, 0)))
        args.append(p_prev)

    return pl.pallas_call(
        body,
        grid=(NP // BN,),
        in_specs=in_specs,
        out_specs=pl.BlockSpec((BN, D), lambda j: (j, 0)),
        out_shape=jax.ShapeDtypeStruct((NP, D), F32),
    )(*args)


def _ln_relu(x_split, p, bl, gl, betl, dinv_n1, want_u):
    BN = 512

    def body(x_ref, p_ref, b_ref, g_ref, bt_ref, dv_ref, o_ref, ou_ref=None):
        xf = jnp.concatenate([x_ref[0], x_ref[1]], axis=-1)
        y = xf + p_ref[...] + b_ref[...]
        m = jnp.mean(y, axis=-1, keepdims=True)
        yc = y - m
        var = jnp.mean(yc * yc, axis=-1, keepdims=True)
        y = yc * (1.0 / jnp.sqrt(var + 1e-5)) * g_ref[...] + bt_ref[...]
        xn = jnp.maximum(y, 0.0)
        o_ref[0] = xn[:, :128]
        o_ref[1] = xn[:, 128:]
        if ou_ref is not None:
            dcol = dv_ref[...]
            ou_ref[0] = xn[:, :128] * dcol
            ou_ref[1] = xn[:, 128:] * dcol

    out_specs = [pl.BlockSpec((NC, BN, 128), lambda j: (0, j, 0))]
    out_shape = [jax.ShapeDtypeStruct((NC, NP, 128), F32)]
    if want_u:
        out_specs.append(pl.BlockSpec((NC, BN, 128), lambda j: (0, j, 0)))
        out_shape.append(jax.ShapeDtypeStruct((NC, NP, 128), F32))

    res = pl.pallas_call(
        body,
        grid=(NP // BN,),
        in_specs=[
            pl.BlockSpec((NC, BN, 128), lambda j: (0, j, 0)),
            pl.BlockSpec((BN, D), lambda j: (j, 0)),
            pl.BlockSpec((1, D), lambda j: (0, 0)),
            pl.BlockSpec((1, D), lambda j: (0, 0)),
            pl.BlockSpec((1, D), lambda j: (0, 0)),
            pl.BlockSpec((BN, 1), lambda j: (j, 0)),
        ],
        out_specs=out_specs,
        out_shape=out_shape,
    )(x_split, p, bl, gl, betl, dinv_n1)
    return res if want_u else (res[0], None)


def kernel(node, edge_index, edge_attr, batch_ptr, W, b, gamma, beta):
    row = edge_index[0].astype(jnp.int32)
    col = edge_index[1].astype(jnp.int32)

    padn = EP - E
    ar = jnp.arange(padn, dtype=jnp.int32)
    row_p = jnp.concatenate([row, (ar * 61) % N]).reshape(EP // CH, CH)
    col_p = jnp.concatenate([col, N + (ar % (NP - N))]).reshape(EP // CH, CH)
    ea_p = jnp.concatenate([edge_attr,
                            jnp.zeros((padn,), F32)]).reshape(EP // CH, CH)

    node_pad = jnp.zeros((NP, D), F32).at[:N].set(node)
    x = jnp.stack([node_pad[:, :128], node_pad[:, 128:]])

    degx = _deg_kernel(col_p, ea_p)
    deg = degx[0, :, 0] + degx[1, :, 0]
    dinv = jnp.where(deg > 0, 1.0 / jnp.sqrt(jnp.where(deg > 0, deg, 1.0)), 0.0)
    dinv2 = dinv * dinv
    dinv_n1 = dinv[:, None]

    u = _prep_u0(x, dinv_n1)

    for l in range(L):
        w2 = [jnp.stack([W[l, k][:, :128].T, W[l, k][:, 128:].T])
              for k in range(K + 1)]
        x1, u1 = _spmm_kernel(u.reshape(NC * NP, 128), ea_p, row_p, col_p,
                              dinv, dinv2)
        p = _mm(x, w2[0], None)
        x2, u2 = _spmm_kernel(u1.reshape(NC * NP, 128), ea_p, row_p, col_p,
                              dinv, dinv2)
        p = _mm(x1, w2[1], p)
        x3, _ = _spmm_kernel(u2.reshape(NC * NP, 128), ea_p, row_p, col_p,
                             dinv, dinv2)
        p = _mm(x2, w2[2], p)
        p = _mm(x3, w2[3], p)
        x, u = _ln_relu(x, p, b[l][None], gamma[l][None], beta[l][None],
                        dinv_n1, want_u=(l < L - 1))

    return jnp.transpose(x, (1, 0, 2)).reshape(NP, D)[:N]
